# Initial kernel scaffold; baseline (speedup 1.0000x reference)
import jax
import jax.numpy as jnp
from jax.experimental import pallas as pl


def kernel(x, edge_index, pred_edges, W1, b1, W2, b2, W3, b3, L1, lb1, L2, lb2, M1, mb1, M2, mb2):
    raise NotImplementedError("write your pallas kernel here")


# speedup vs baseline: 14.2269x; 14.2269x over previous
import functools

import jax
import jax.numpy as jnp
from jax import lax
from jax.experimental import pallas as pl
from jax.experimental.pallas import tpu as pltpu
from jax.experimental.pallas import tpu_sc as plsc

N = 10000
E = 320000
P = 100000
NW = 32
EW = E // NW
CH = 125
NCH = EW // CH
NPAD = 10240
RT = NPAD // 16
ZB = 128
PCH = 128
PNCH = 25
PPAD = NW * PNCH * PCH

_mesh = plsc.VectorSubcoreMesh(core_axis_name="c", subcore_axis_name="s")


def _deg_sc(dst_r):
    @functools.partial(
        pl.kernel, mesh=_mesh,
        compiler_params=pltpu.CompilerParams(use_tc_tiling_on_sc=False),
        out_type=jax.ShapeDtypeStruct((2, NPAD, 16), jnp.float32),
        scratch_types=[
            pltpu.VMEM((NCH, CH), jnp.int32),
            pltpu.VMEM((CH, 16), jnp.float32),
            pltpu.VMEM((ZB, 16), jnp.float32),
            pltpu.VMEM_SHARED((NPAD, 16), jnp.float32),
        ],
    )
    def k(dstr_hbm, out_hbm, dstv, ones_v, zero_v, acc):
        c = lax.axis_index("c")
        s = lax.axis_index("s")
        wid = s * 2 + c
        pltpu.sync_copy(dstr_hbm.at[wid], dstv)

        def fill(i, _):
            ones_v[i] = jnp.full((16,), 1.0, jnp.float32)
            return 0
        lax.fori_loop(0, CH, fill, 0)

        def zfill(i, _):
            zero_v[i] = jnp.zeros((16,), jnp.float32)
            return 0
        lax.fori_loop(0, ZB, zfill, 0)
        for z in range(RT // ZB):
            pltpu.sync_copy(zero_v, acc.at[pl.ds(s * RT + z * ZB, ZB)])
        plsc.subcore_barrier()

        def chunk(j, _):
            pltpu.sync_copy(ones_v, acc.at[dstv.at[j]], add=True)
            return 0
        lax.fori_loop(0, NCH, chunk, 0)
        plsc.subcore_barrier()
        pltpu.sync_copy(acc.at[pl.ds(s * RT, RT)], out_hbm.at[c, pl.ds(s * RT, RT)])

    return k(dst_r)


def _scatter_sc(g, src_r, dst_r):
    D = g.shape[1]

    @functools.partial(
        pl.kernel, mesh=_mesh,
        compiler_params=pltpu.CompilerParams(use_tc_tiling_on_sc=False),
        out_type=jax.ShapeDtypeStruct((2, NPAD, D), jnp.float32),
        scratch_types=[
            pltpu.VMEM((NCH, CH), jnp.int32),
            pltpu.VMEM((NCH, CH), jnp.int32),
            pltpu.VMEM((CH, D), jnp.float32),
            pltpu.VMEM((ZB, D), jnp.float32),
            pltpu.VMEM_SHARED((NPAD, D), jnp.float32),
        ],
    )
    def k(g_hbm, srcr_hbm, dstr_hbm, out_hbm, srcv, dstv, rows, zero_v, acc):
        c = lax.axis_index("c")
        s = lax.axis_index("s")
        wid = s * 2 + c
        pltpu.sync_copy(srcr_hbm.at[wid], srcv)
        pltpu.sync_copy(dstr_hbm.at[wid], dstv)

        nsub = D // 16

        def zrow(t, _):
            zero_v[t // nsub, pl.ds((t % nsub) * 16, 16)] = jnp.zeros(
                (16,), jnp.float32)
            return 0
        lax.fori_loop(0, ZB * nsub, zrow, 0)
        for z in range(RT // ZB):
            pltpu.sync_copy(zero_v, acc.at[pl.ds(s * RT + z * ZB, ZB)])
        plsc.subcore_barrier()

        def chunk(j, _):
            pltpu.sync_copy(g_hbm.at[srcv.at[j]], rows)
            pltpu.sync_copy(rows, acc.at[dstv.at[j]], add=True)
            return 0
        lax.fori_loop(0, NCH, chunk, 0)
        plsc.subcore_barrier()
        pltpu.sync_copy(acc.at[pl.ds(s * RT, RT)], out_hbm.at[c, pl.ds(s * RT, RT)])

    return k(g, src_r, dst_r)


def _pairgather_sc(ta, tb, u_r, v_r):
    @functools.partial(
        pl.kernel, mesh=_mesh,
        compiler_params=pltpu.CompilerParams(use_tc_tiling_on_sc=False),
        out_type=(jax.ShapeDtypeStruct((PPAD, 16), jnp.float32),
                  jax.ShapeDtypeStruct((PPAD, 16), jnp.float32)),
        scratch_types=[
            pltpu.VMEM((PNCH, PCH), jnp.int32),
            pltpu.VMEM((PNCH, PCH), jnp.int32),
            pltpu.VMEM((PCH, 16), jnp.float32),
            pltpu.VMEM((PCH, 16), jnp.float32),
        ],
    )
    def k(ta_hbm, tb_hbm, ur_hbm, vr_hbm, outa_hbm, outb_hbm, uv, vv, bufa, bufb):
        c = lax.axis_index("c")
        s = lax.axis_index("s")
        wid = s * 2 + c
        pltpu.sync_copy(ur_hbm.at[wid], uv)
        pltpu.sync_copy(vr_hbm.at[wid], vv)

        def chunk(j, _):
            base = (wid * PNCH + j) * PCH
            pltpu.sync_copy(ta_hbm.at[uv.at[j]], bufa)
            pltpu.sync_copy(bufa, outa_hbm.at[pl.ds(base, PCH)])
            pltpu.sync_copy(tb_hbm.at[vv.at[j]], bufb)
            pltpu.sync_copy(bufb, outb_hbm.at[pl.ds(base, PCH)])
            return 0
        lax.fori_loop(0, PNCH, chunk, 0)

    return k(ta, tb, u_r, v_r)


_BLK = 1000


def _tc_first(x, w1, d0, d1):
    def body(x_ref, w_ref, d0_ref, d1_ref, ga_ref, gb_ref, dis_ref):
        deg = d0_ref[:, 0:1] + d1_ref[:, 0:1] + 1.0
        dis = lax.rsqrt(deg)
        h = jnp.dot(x_ref[...], w_ref[...], preferred_element_type=jnp.float32)
        g = dis * h
        ga_ref[...] = g[:, :64]
        gb_ref[...] = g[:, 64:]
        dis_ref[...] = jnp.broadcast_to(dis, (_BLK, 16))

    return pl.pallas_call(
        body,
        grid=(N // _BLK,),
        in_specs=[pl.BlockSpec((_BLK, 128), lambda i: (i, 0)),
                  pl.BlockSpec((128, 128), lambda i: (0, 0)),
                  pl.BlockSpec((_BLK, 16), lambda i: (i, 0)),
                  pl.BlockSpec((_BLK, 16), lambda i: (i, 0))],
        out_specs=[pl.BlockSpec((_BLK, 64), lambda i: (i, 0)),
                   pl.BlockSpec((_BLK, 64), lambda i: (i, 0)),
                   pl.BlockSpec((_BLK, 16), lambda i: (i, 0))],
        out_shape=[jax.ShapeDtypeStruct((N, 64), jnp.float32),
                   jax.ShapeDtypeStruct((N, 64), jnp.float32),
                   jax.ShapeDtypeStruct((N, 16), jnp.float32)],
    )(x, w1, d0, d1)


def _tc_mid2(p0a, p1a, p0b, p1b, ga, gb, dis16, ba_row, bb_row, wa, wb):
    def body(p0a_ref, p1a_ref, p0b_ref, p1b_ref, ga_ref, gb_ref, dis_ref,
             ba_ref, bb_ref, wa_ref, wb_ref, out_ref):
        dis = dis_ref[:, 0:1]
        t_a = jnp.maximum(
            dis * (p0a_ref[...] + p1a_ref[...] + ga_ref[...]) + ba_ref[...],
            0.0)
        t_b = jnp.maximum(
            dis * (p0b_ref[...] + p1b_ref[...] + gb_ref[...]) + bb_ref[...],
            0.0)
        out_ref[...] = dis * (
            jnp.dot(t_a, wa_ref[...], preferred_element_type=jnp.float32)
            + jnp.dot(t_b, wb_ref[...], preferred_element_type=jnp.float32))

    blk64 = pl.BlockSpec((_BLK, 64), lambda i: (i, 0))
    return pl.pallas_call(
        body,
        grid=(N // _BLK,),
        in_specs=[blk64, blk64, blk64, blk64, blk64, blk64,
                  pl.BlockSpec((_BLK, 16), lambda i: (i, 0)),
                  pl.BlockSpec((1, 64), lambda i: (0, 0)),
                  pl.BlockSpec((1, 64), lambda i: (0, 0)),
                  pl.BlockSpec((64, 64), lambda i: (0, 0)),
                  pl.BlockSpec((64, 64), lambda i: (0, 0))],
        out_specs=pl.BlockSpec((_BLK, 64), lambda i: (i, 0)),
        out_shape=jax.ShapeDtypeStruct((N, 64), jnp.float32),
    )(p0a, p1a, p0b, p1b, ga, gb, dis16, ba_row, bb_row, wa, wb)


def _tc_mid(p0, p1, g, dis16, b_row, w):
    din = g.shape[1]
    dout = w.shape[1]

    def body(p0_ref, p1_ref, g_ref, dis_ref, b_ref, w_ref, out_ref):
        dis = dis_ref[:, 0:1]
        t = dis * (p0_ref[...] + p1_ref[...] + g_ref[...]) + b_ref[...]
        t = jnp.maximum(t, 0.0)
        out_ref[...] = dis * jnp.dot(t, w_ref[...],
                                     preferred_element_type=jnp.float32)

    return pl.pallas_call(
        body,
        grid=(N // _BLK,),
        in_specs=[pl.BlockSpec((_BLK, din), lambda i: (i, 0)),
                  pl.BlockSpec((_BLK, din), lambda i: (i, 0)),
                  pl.BlockSpec((_BLK, din), lambda i: (i, 0)),
                  pl.BlockSpec((_BLK, 16), lambda i: (i, 0)),
                  pl.BlockSpec((1, din), lambda i: (0, 0)),
                  pl.BlockSpec((din, dout), lambda i: (0, 0))],
        out_specs=pl.BlockSpec((_BLK, dout), lambda i: (i, 0)),
        out_shape=jax.ShapeDtypeStruct((N, dout), jnp.float32),
    )(p0, p1, g, dis16, b_row, w)


def _tc_head(p0, p1, g3, dis16, b3_row, l1, lb1_row, l2, lb2_row, m1, mb1_row):
    def body(p0_ref, p1_ref, g_ref, dis_ref, b3_ref, l1_ref, lb1_ref,
             l2_ref, lb2_ref, m1_ref, mb1_ref, ta_ref, tb_ref):
        dis = dis_ref[:, 0:1]
        o = dis * (p0_ref[...] + p1_ref[...] + g_ref[...]) + b3_ref[...]
        o = jnp.maximum(o, 0.0)
        h4 = jnp.maximum(
            jnp.dot(o, l1_ref[...], preferred_element_type=jnp.float32)
            + lb1_ref[...], 0.0)
        emb = jnp.maximum(
            jnp.dot(h4, l2_ref[...], preferred_element_type=jnp.float32)
            + lb2_ref[...], 0.0)
        m1 = m1_ref[...]
        a = jnp.dot(emb, m1[:16, :], preferred_element_type=jnp.float32) \
            + mb1_ref[...]
        b = jnp.dot(emb, m1[16:, :], preferred_element_type=jnp.float32)
        ta_ref[...] = jnp.concatenate([a, a], axis=1)
        tb_ref[...] = jnp.concatenate([b, b], axis=1)

    return pl.pallas_call(
        body,
        grid=(N // _BLK,),
        in_specs=[pl.BlockSpec((_BLK, 32), lambda i: (i, 0)),
                  pl.BlockSpec((_BLK, 32), lambda i: (i, 0)),
                  pl.BlockSpec((_BLK, 32), lambda i: (i, 0)),
                  pl.BlockSpec((_BLK, 16), lambda i: (i, 0)),
                  pl.BlockSpec((1, 32), lambda i: (0, 0)),
                  pl.BlockSpec((32, 16), lambda i: (0, 0)),
                  pl.BlockSpec((1, 16), lambda i: (0, 0)),
                  pl.BlockSpec((16, 16), lambda i: (0, 0)),
                  pl.BlockSpec((1, 16), lambda i: (0, 0)),
                  pl.BlockSpec((32, 8), lambda i: (0, 0)),
                  pl.BlockSpec((1, 8), lambda i: (0, 0))],
        out_specs=[pl.BlockSpec((_BLK, 16), lambda i: (i, 0)),
                   pl.BlockSpec((_BLK, 16), lambda i: (i, 0))],
        out_shape=[jax.ShapeDtypeStruct((N, 16), jnp.float32),
                   jax.ShapeDtypeStruct((N, 16), jnp.float32)],
    )(p0, p1, g3, dis16, b3_row, l1, lb1_row, l2, lb2_row, m1, mb1_row)


def _tc_final(ga, gb, m2_pat, mb2_s):
    blk = 1024

    def body(a_ref, b_ref, m2_ref, mb2_ref, out_ref):
        t = jnp.maximum(a_ref[...] + b_ref[...], 0.0)
        sc = jnp.sum(t * m2_ref[...], axis=1, keepdims=True) + mb2_ref[...]
        y = 1.0 / (1.0 + jnp.exp(-sc))
        out_ref[...] = jnp.broadcast_to(y, (blk, 8))

    return pl.pallas_call(
        body,
        grid=(PPAD // blk,),
        in_specs=[pl.BlockSpec((blk, 16), lambda i: (i, 0)),
                  pl.BlockSpec((blk, 16), lambda i: (i, 0)),
                  pl.BlockSpec((1, 16), lambda i: (0, 0)),
                  pl.BlockSpec((1, 1), lambda i: (0, 0))],
        out_specs=pl.BlockSpec((blk, 8), lambda i: (i, 0)),
        out_shape=jax.ShapeDtypeStruct((PPAD, 8), jnp.float32),
    )(ga, gb, m2_pat, mb2_s)


def kernel(x, edge_index, pred_edges, W1, b1, W2, b2, W3, b3,
           L1, lb1, L2, lb2, M1, mb1, M2, mb2):
    ei = edge_index.astype(jnp.int32)
    src_r = ei[0].reshape(NW, NCH, CH)
    dst_r = ei[1].reshape(NW, NCH, CH)
    pe = pred_edges.astype(jnp.int32)
    u_r = jnp.pad(pe[:, 0], (0, PPAD - P)).reshape(NW, PNCH, PCH)
    v_r = jnp.pad(pe[:, 1], (0, PPAD - P)).reshape(NW, PNCH, PCH)

    degp = _deg_sc(dst_r)
    g1a, g1b, dis16 = _tc_first(x, W1, degp[0, :N], degp[1, :N])

    ppa = _scatter_sc(g1a, src_r, dst_r)
    ppb = _scatter_sc(g1b, src_r, dst_r)
    g2 = _tc_mid2(ppa[0, :N], ppa[1, :N], ppb[0, :N], ppb[1, :N],
                  g1a, g1b, dis16,
                  b1[:64].reshape(1, -1), b1[64:].reshape(1, -1),
                  W2[:64], W2[64:])

    pp = _scatter_sc(g2, src_r, dst_r)
    g3 = _tc_mid(pp[0, :N], pp[1, :N], g2, dis16, b2.reshape(1, -1), W3)

    pp = _scatter_sc(g3, src_r, dst_r)
    ta, tb = _tc_head(pp[0, :N], pp[1, :N], g3, dis16, b3.reshape(1, -1),
                      L1, lb1.reshape(1, -1), L2, lb2.reshape(1, -1),
                      M1, mb1.reshape(1, -1))

    ga, gb = _pairgather_sc(ta, tb, u_r, v_r)
    m2_pat = jnp.concatenate([M2[:, 0], jnp.zeros((8,), jnp.float32)])
    y = _tc_final(ga, gb, m2_pat.reshape(1, 16), mb2.reshape(1, 1))
    return y[:P, 0]


# speedup vs baseline: 16.7415x; 1.1768x over previous
import functools

import jax
import jax.numpy as jnp
from jax import lax
from jax.experimental import pallas as pl
from jax.experimental.pallas import tpu as pltpu
from jax.experimental.pallas import tpu_sc as plsc

N = 10000
E = 320000
P = 100000
NW = 32
EW = E // NW
CH = 125
NCH = EW // CH
NPAD = 10240
RT = NPAD // 16
ZB = 128
PCH = 100
PNCH = 32
PPAD = NW * PNCH * PCH

_mesh = plsc.VectorSubcoreMesh(core_axis_name="c", subcore_axis_name="s")


def _deg_sc(dst_r):
    @functools.partial(
        pl.kernel, mesh=_mesh,
        compiler_params=pltpu.CompilerParams(use_tc_tiling_on_sc=False),
        out_type=jax.ShapeDtypeStruct((2, NPAD, 16), jnp.float32),
        scratch_types=[
            pltpu.VMEM((NCH, CH), jnp.int32),
            pltpu.VMEM((CH, 16), jnp.float32),
            pltpu.VMEM((ZB, 16), jnp.float32),
            pltpu.VMEM_SHARED((NPAD, 16), jnp.float32),
        ],
    )
    def k(dstr_hbm, out_hbm, dstv, ones_v, zero_v, acc):
        c = lax.axis_index("c")
        s = lax.axis_index("s")
        wid = s * 2 + c
        pltpu.sync_copy(dstr_hbm.at[wid], dstv)

        def fill(i, _):
            ones_v[i] = jnp.full((16,), 1.0, jnp.float32)
            return 0
        lax.fori_loop(0, CH, fill, 0)

        def zfill(i, _):
            zero_v[i] = jnp.zeros((16,), jnp.float32)
            return 0
        lax.fori_loop(0, ZB, zfill, 0)
        for z in range(RT // ZB):
            pltpu.sync_copy(zero_v, acc.at[pl.ds(s * RT + z * ZB, ZB)])
        plsc.subcore_barrier()

        def chunk(j, _):
            pltpu.sync_copy(ones_v, acc.at[dstv.at[j]], add=True)
            return 0
        lax.fori_loop(0, NCH, chunk, 0)
        plsc.subcore_barrier()
        pltpu.sync_copy(acc.at[pl.ds(s * RT, RT)], out_hbm.at[c, pl.ds(s * RT, RT)])

    return k(dst_r)


def _scatter_sc(g, src_r, dst_r):
    D = g.shape[1]

    @functools.partial(
        pl.kernel, mesh=_mesh,
        compiler_params=pltpu.CompilerParams(use_tc_tiling_on_sc=False),
        out_type=jax.ShapeDtypeStruct((2, NPAD, D), jnp.float32),
        scratch_types=[
            pltpu.VMEM((NCH, CH), jnp.int32),
            pltpu.VMEM((NCH, CH), jnp.int32),
            pltpu.VMEM((CH, D), jnp.float32),
            pltpu.VMEM((CH, D), jnp.float32),
            pltpu.VMEM((ZB, D), jnp.float32),
            pltpu.VMEM_SHARED((NPAD, D), jnp.float32),
            pltpu.SemaphoreType.DMA,
            pltpu.SemaphoreType.DMA,
        ],
    )
    def k(g_hbm, srcr_hbm, dstr_hbm, out_hbm, srcv, dstv, rows0, rows1,
          zero_v, acc, sem0, sem1):
        c = lax.axis_index("c")
        s = lax.axis_index("s")
        wid = s * 2 + c
        pltpu.sync_copy(srcr_hbm.at[wid], srcv)
        pltpu.sync_copy(dstr_hbm.at[wid], dstv)

        nsub = D // 16

        def zrow(t, _):
            zero_v[t // nsub, pl.ds((t % nsub) * 16, 16)] = jnp.zeros(
                (16,), jnp.float32)
            return 0
        lax.fori_loop(0, ZB * nsub, zrow, 0)
        for z in range(RT // ZB):
            pltpu.sync_copy(zero_v, acc.at[pl.ds(s * RT + z * ZB, ZB)])
        plsc.subcore_barrier()

        pltpu.async_copy(g_hbm.at[srcv.at[0]], rows0, sem0)

        def chunk2(jj, _):
            j0 = 2 * jj
            pltpu.make_async_copy(g_hbm.at[srcv.at[j0]], rows0, sem0).wait()
            pltpu.async_copy(g_hbm.at[srcv.at[j0 + 1]], rows1, sem1)
            pltpu.sync_copy(rows0, acc.at[dstv.at[j0]], add=True)
            pltpu.make_async_copy(g_hbm.at[srcv.at[j0 + 1]], rows1,
                                  sem1).wait()

            @pl.when(jj + 1 < NCH // 2)
            def _():
                pltpu.async_copy(g_hbm.at[srcv.at[j0 + 2]], rows0, sem0)

            pltpu.sync_copy(rows1, acc.at[dstv.at[j0 + 1]], add=True)
            return 0
        lax.fori_loop(0, NCH // 2, chunk2, 0)
        plsc.subcore_barrier()
        pltpu.sync_copy(acc.at[pl.ds(s * RT, RT)], out_hbm.at[c, pl.ds(s * RT, RT)])

    return k(g, src_r, dst_r)


def _pairgather_sc(ta, tb, u_r, v_r):
    @functools.partial(
        pl.kernel, mesh=_mesh,
        compiler_params=pltpu.CompilerParams(use_tc_tiling_on_sc=False),
        out_type=(jax.ShapeDtypeStruct((PPAD, 16), jnp.float32),
                  jax.ShapeDtypeStruct((PPAD, 16), jnp.float32)),
        scratch_types=[
            pltpu.VMEM((PNCH, PCH), jnp.int32),
            pltpu.VMEM((PNCH, PCH), jnp.int32),
            pltpu.VMEM((PCH, 16), jnp.float32),
            pltpu.VMEM((PCH, 16), jnp.float32),
            pltpu.VMEM((PCH, 16), jnp.float32),
            pltpu.VMEM((PCH, 16), jnp.float32),
            pltpu.SemaphoreType.DMA,
            pltpu.SemaphoreType.DMA,
            pltpu.SemaphoreType.DMA,
            pltpu.SemaphoreType.DMA,
        ],
    )
    def k(ta_hbm, tb_hbm, ur_hbm, vr_hbm, outa_hbm, outb_hbm, uv, vv,
          bufa0, bufb0, bufa1, bufb1, sa0, sb0, sa1, sb1):
        c = lax.axis_index("c")
        s = lax.axis_index("s")
        wid = s * 2 + c
        pltpu.sync_copy(ur_hbm.at[wid], uv)
        pltpu.sync_copy(vr_hbm.at[wid], vv)

        pltpu.async_copy(ta_hbm.at[uv.at[0]], bufa0, sa0)
        pltpu.async_copy(tb_hbm.at[vv.at[0]], bufb0, sb0)
        pltpu.async_copy(ta_hbm.at[uv.at[1]], bufa1, sa1)
        pltpu.async_copy(tb_hbm.at[vv.at[1]], bufb1, sb1)

        def chunk2(jj, _):
            j0 = 2 * jj
            for (j, ba, bb, sba, sbb) in ((j0, bufa0, bufb0, sa0, sb0),
                                          (j0 + 1, bufa1, bufb1, sa1, sb1)):
                base = (wid * PNCH + j) * PCH
                pltpu.make_async_copy(ta_hbm.at[uv.at[j]], ba, sba).wait()
                pltpu.make_async_copy(tb_hbm.at[vv.at[j]], bb, sbb).wait()
                pltpu.sync_copy(ba, outa_hbm.at[pl.ds(base, PCH)])
                pltpu.sync_copy(bb, outb_hbm.at[pl.ds(base, PCH)])

                @pl.when(j + 2 < PNCH)
                def _():
                    pltpu.async_copy(ta_hbm.at[uv.at[j + 2]], ba, sba)
                    pltpu.async_copy(tb_hbm.at[vv.at[j + 2]], bb, sbb)
            return 0
        lax.fori_loop(0, PNCH // 2, chunk2, 0)

    return k(ta, tb, u_r, v_r)


_BLK = 1000


def _tc_first(x, w1, d0, d1):
    def body(x_ref, w_ref, d0_ref, d1_ref, ga_ref, gb_ref, dis_ref):
        deg = d0_ref[:, 0:1] + d1_ref[:, 0:1] + 1.0
        dis = lax.rsqrt(deg)
        h = jnp.dot(x_ref[...], w_ref[...], preferred_element_type=jnp.float32)
        g = dis * h
        ga_ref[...] = g[:, :64]
        gb_ref[...] = g[:, 64:]
        dis_ref[...] = jnp.broadcast_to(dis, (_BLK, 16))

    return pl.pallas_call(
        body,
        grid=(N // _BLK,),
        in_specs=[pl.BlockSpec((_BLK, 128), lambda i: (i, 0)),
                  pl.BlockSpec((128, 128), lambda i: (0, 0)),
                  pl.BlockSpec((_BLK, 16), lambda i: (i, 0)),
                  pl.BlockSpec((_BLK, 16), lambda i: (i, 0))],
        out_specs=[pl.BlockSpec((_BLK, 64), lambda i: (i, 0)),
                   pl.BlockSpec((_BLK, 64), lambda i: (i, 0)),
                   pl.BlockSpec((_BLK, 16), lambda i: (i, 0))],
        out_shape=[jax.ShapeDtypeStruct((N, 64), jnp.float32),
                   jax.ShapeDtypeStruct((N, 64), jnp.float32),
                   jax.ShapeDtypeStruct((N, 16), jnp.float32)],
    )(x, w1, d0, d1)


def _tc_mid2(p0a, p1a, p0b, p1b, ga, gb, dis16, ba_row, bb_row, wa, wb):
    def body(p0a_ref, p1a_ref, p0b_ref, p1b_ref, ga_ref, gb_ref, dis_ref,
             ba_ref, bb_ref, wa_ref, wb_ref, out_ref):
        dis = dis_ref[:, 0:1]
        t_a = jnp.maximum(
            dis * (p0a_ref[...] + p1a_ref[...] + ga_ref[...]) + ba_ref[...],
            0.0)
        t_b = jnp.maximum(
            dis * (p0b_ref[...] + p1b_ref[...] + gb_ref[...]) + bb_ref[...],
            0.0)
        out_ref[...] = dis * (
            jnp.dot(t_a, wa_ref[...], preferred_element_type=jnp.float32)
            + jnp.dot(t_b, wb_ref[...], preferred_element_type=jnp.float32))

    blk64 = pl.BlockSpec((_BLK, 64), lambda i: (i, 0))
    return pl.pallas_call(
        body,
        grid=(N // _BLK,),
        in_specs=[blk64, blk64, blk64, blk64, blk64, blk64,
                  pl.BlockSpec((_BLK, 16), lambda i: (i, 0)),
                  pl.BlockSpec((1, 64), lambda i: (0, 0)),
                  pl.BlockSpec((1, 64), lambda i: (0, 0)),
                  pl.BlockSpec((64, 64), lambda i: (0, 0)),
                  pl.BlockSpec((64, 64), lambda i: (0, 0))],
        out_specs=pl.BlockSpec((_BLK, 64), lambda i: (i, 0)),
        out_shape=jax.ShapeDtypeStruct((N, 64), jnp.float32),
    )(p0a, p1a, p0b, p1b, ga, gb, dis16, ba_row, bb_row, wa, wb)


def _tc_mid(p0, p1, g, dis16, b_row, w):
    din = g.shape[1]
    dout = w.shape[1]

    def body(p0_ref, p1_ref, g_ref, dis_ref, b_ref, w_ref, out_ref):
        dis = dis_ref[:, 0:1]
        t = dis * (p0_ref[...] + p1_ref[...] + g_ref[...]) + b_ref[...]
        t = jnp.maximum(t, 0.0)
        out_ref[...] = dis * jnp.dot(t, w_ref[...],
                                     preferred_element_type=jnp.float32)

    return pl.pallas_call(
        body,
        grid=(N // _BLK,),
        in_specs=[pl.BlockSpec((_BLK, din), lambda i: (i, 0)),
                  pl.BlockSpec((_BLK, din), lambda i: (i, 0)),
                  pl.BlockSpec((_BLK, din), lambda i: (i, 0)),
                  pl.BlockSpec((_BLK, 16), lambda i: (i, 0)),
                  pl.BlockSpec((1, din), lambda i: (0, 0)),
                  pl.BlockSpec((din, dout), lambda i: (0, 0))],
        out_specs=pl.BlockSpec((_BLK, dout), lambda i: (i, 0)),
        out_shape=jax.ShapeDtypeStruct((N, dout), jnp.float32),
    )(p0, p1, g, dis16, b_row, w)


def _tc_head(p0, p1, g3, dis16, b3_row, l1, lb1_row, l2, lb2_row, m1, mb1_row):
    def body(p0_ref, p1_ref, g_ref, dis_ref, b3_ref, l1_ref, lb1_ref,
             l2_ref, lb2_ref, m1_ref, mb1_ref, ta_ref, tb_ref):
        dis = dis_ref[:, 0:1]
        o = dis * (p0_ref[...] + p1_ref[...] + g_ref[...]) + b3_ref[...]
        o = jnp.maximum(o, 0.0)
        h4 = jnp.maximum(
            jnp.dot(o, l1_ref[...], preferred_element_type=jnp.float32)
            + lb1_ref[...], 0.0)
        emb = jnp.maximum(
            jnp.dot(h4, l2_ref[...], preferred_element_type=jnp.float32)
            + lb2_ref[...], 0.0)
        m1 = m1_ref[...]
        a = jnp.dot(emb, m1[:16, :], preferred_element_type=jnp.float32) \
            + mb1_ref[...]
        b = jnp.dot(emb, m1[16:, :], preferred_element_type=jnp.float32)
        ta_ref[...] = jnp.concatenate([a, a], axis=1)
        tb_ref[...] = jnp.concatenate([b, b], axis=1)

    return pl.pallas_call(
        body,
        grid=(N // _BLK,),
        in_specs=[pl.BlockSpec((_BLK, 32), lambda i: (i, 0)),
                  pl.BlockSpec((_BLK, 32), lambda i: (i, 0)),
                  pl.BlockSpec((_BLK, 32), lambda i: (i, 0)),
                  pl.BlockSpec((_BLK, 16), lambda i: (i, 0)),
                  pl.BlockSpec((1, 32), lambda i: (0, 0)),
                  pl.BlockSpec((32, 16), lambda i: (0, 0)),
                  pl.BlockSpec((1, 16), lambda i: (0, 0)),
                  pl.BlockSpec((16, 16), lambda i: (0, 0)),
                  pl.BlockSpec((1, 16), lambda i: (0, 0)),
                  pl.BlockSpec((32, 8), lambda i: (0, 0)),
                  pl.BlockSpec((1, 8), lambda i: (0, 0))],
        out_specs=[pl.BlockSpec((_BLK, 16), lambda i: (i, 0)),
                   pl.BlockSpec((_BLK, 16), lambda i: (i, 0))],
        out_shape=[jax.ShapeDtypeStruct((N, 16), jnp.float32),
                   jax.ShapeDtypeStruct((N, 16), jnp.float32)],
    )(p0, p1, g3, dis16, b3_row, l1, lb1_row, l2, lb2_row, m1, mb1_row)


def _tc_final(ga, gb, m2_pat, mb2_s):
    blk = 1024

    def body(a_ref, b_ref, m2_ref, mb2_ref, out_ref):
        t = jnp.maximum(a_ref[...] + b_ref[...], 0.0)
        sc = jnp.sum(t * m2_ref[...], axis=1, keepdims=True) + mb2_ref[...]
        y = 1.0 / (1.0 + jnp.exp(-sc))
        out_ref[...] = jnp.broadcast_to(y, (blk, 8))

    return pl.pallas_call(
        body,
        grid=(PPAD // blk,),
        in_specs=[pl.BlockSpec((blk, 16), lambda i: (i, 0)),
                  pl.BlockSpec((blk, 16), lambda i: (i, 0)),
                  pl.BlockSpec((1, 16), lambda i: (0, 0)),
                  pl.BlockSpec((1, 1), lambda i: (0, 0))],
        out_specs=pl.BlockSpec((blk, 8), lambda i: (i, 0)),
        out_shape=jax.ShapeDtypeStruct((PPAD, 8), jnp.float32),
    )(ga, gb, m2_pat, mb2_s)


def kernel(x, edge_index, pred_edges, W1, b1, W2, b2, W3, b3,
           L1, lb1, L2, lb2, M1, mb1, M2, mb2):
    ei = edge_index.astype(jnp.int32)
    src_r = ei[0].reshape(NW, NCH, CH)
    dst_r = ei[1].reshape(NW, NCH, CH)
    pe = pred_edges.astype(jnp.int32)
    u_r = jnp.pad(pe[:, 0], (0, PPAD - P)).reshape(NW, PNCH, PCH)
    v_r = jnp.pad(pe[:, 1], (0, PPAD - P)).reshape(NW, PNCH, PCH)

    degp = _deg_sc(dst_r)
    g1a, g1b, dis16 = _tc_first(x, W1, degp[0, :N], degp[1, :N])

    ppa = _scatter_sc(g1a, src_r, dst_r)
    ppb = _scatter_sc(g1b, src_r, dst_r)
    g2 = _tc_mid2(ppa[0, :N], ppa[1, :N], ppb[0, :N], ppb[1, :N],
                  g1a, g1b, dis16,
                  b1[:64].reshape(1, -1), b1[64:].reshape(1, -1),
                  W2[:64], W2[64:])

    pp = _scatter_sc(g2, src_r, dst_r)
    g3 = _tc_mid(pp[0, :N], pp[1, :N], g2, dis16, b2.reshape(1, -1), W3)

    pp = _scatter_sc(g3, src_r, dst_r)
    ta, tb = _tc_head(pp[0, :N], pp[1, :N], g3, dis16, b3.reshape(1, -1),
                      L1, lb1.reshape(1, -1), L2, lb2.reshape(1, -1),
                      M1, mb1.reshape(1, -1))

    ga, gb = _pairgather_sc(ta, tb, u_r, v_r)
    m2_pat = jnp.concatenate([M2[:, 0], jnp.zeros((8,), jnp.float32)])
    y = _tc_final(ga, gb, m2_pat.reshape(1, 16), mb2.reshape(1, 1))
    return y[:P, 0]


# speedup vs baseline: 21.0928x; 1.2599x over previous
import functools

import jax
import jax.numpy as jnp
from jax import lax
from jax.experimental import pallas as pl
from jax.experimental.pallas import tpu as pltpu
from jax.experimental.pallas import tpu_sc as plsc

N = 10000
E = 320000
P = 100000
NW = 32
EW = E // NW
CH = 125
NCH = EW // CH
NPAD = 10240
RT = NPAD // 16
ZB = 128
PCH = 100
PNCH = 32
PPAD = NW * PNCH * PCH

_mesh = plsc.VectorSubcoreMesh(core_axis_name="c", subcore_axis_name="s")


def _deg_sc(dst_r):
    @functools.partial(
        pl.kernel, mesh=_mesh,
        compiler_params=pltpu.CompilerParams(use_tc_tiling_on_sc=False),
        out_type=jax.ShapeDtypeStruct((2, NPAD, 16), jnp.float32),
        scratch_types=[
            pltpu.VMEM((NCH, CH), jnp.int32),
            pltpu.VMEM((CH, 16), jnp.float32),
            pltpu.VMEM((ZB, 16), jnp.float32),
            pltpu.VMEM_SHARED((NPAD, 16), jnp.float32),
        ],
    )
    def k(dstr_hbm, out_hbm, dstv, ones_v, zero_v, acc):
        c = lax.axis_index("c")
        s = lax.axis_index("s")
        wid = s * 2 + c
        pltpu.sync_copy(dstr_hbm.at[wid], dstv)

        def fill(i, _):
            ones_v[i] = jnp.full((16,), 1.0, jnp.float32)
            return 0
        lax.fori_loop(0, CH, fill, 0)

        def zfill(i, _):
            zero_v[i] = jnp.zeros((16,), jnp.float32)
            return 0
        lax.fori_loop(0, ZB, zfill, 0)
        for z in range(RT // ZB):
            pltpu.sync_copy(zero_v, acc.at[pl.ds(s * RT + z * ZB, ZB)])
        plsc.subcore_barrier()

        def chunk(j, _):
            pltpu.sync_copy(ones_v, acc.at[dstv.at[j]], add=True)
            return 0
        lax.fori_loop(0, NCH, chunk, 0)
        plsc.subcore_barrier()
        pltpu.sync_copy(acc.at[pl.ds(s * RT, RT)], out_hbm.at[c, pl.ds(s * RT, RT)])

    return k(dst_r)


def _scatter_sc(g, src_r, dst_r):
    D = g.shape[1]

    @functools.partial(
        pl.kernel, mesh=_mesh,
        compiler_params=pltpu.CompilerParams(use_tc_tiling_on_sc=False),
        out_type=jax.ShapeDtypeStruct((2, NPAD, D), jnp.float32),
        scratch_types=[
            pltpu.VMEM((NCH, CH), jnp.int32),
            pltpu.VMEM((NCH, CH), jnp.int32),
            pltpu.VMEM((CH, D), jnp.float32),
            pltpu.VMEM((CH, D), jnp.float32),
            pltpu.VMEM((ZB, D), jnp.float32),
            pltpu.VMEM_SHARED((NPAD, D), jnp.float32),
            pltpu.SemaphoreType.DMA,
            pltpu.SemaphoreType.DMA,
        ],
    )
    def k(g_hbm, srcr_hbm, dstr_hbm, out_hbm, srcv, dstv, rows0, rows1,
          zero_v, acc, sem0, sem1):
        c = lax.axis_index("c")
        s = lax.axis_index("s")
        wid = s * 2 + c
        pltpu.sync_copy(srcr_hbm.at[wid], srcv)
        pltpu.sync_copy(dstr_hbm.at[wid], dstv)

        nsub = D // 16

        def zrow(t, _):
            zero_v[t // nsub, pl.ds((t % nsub) * 16, 16)] = jnp.zeros(
                (16,), jnp.float32)
            return 0
        lax.fori_loop(0, ZB * nsub, zrow, 0)
        for z in range(RT // ZB):
            pltpu.sync_copy(zero_v, acc.at[pl.ds(s * RT + z * ZB, ZB)])
        plsc.subcore_barrier()

        pltpu.async_copy(g_hbm.at[srcv.at[0]], rows0, sem0)

        def chunk2(jj, _):
            j0 = 2 * jj
            pltpu.make_async_copy(g_hbm.at[srcv.at[j0]], rows0, sem0).wait()
            pltpu.async_copy(g_hbm.at[srcv.at[j0 + 1]], rows1, sem1)
            pltpu.sync_copy(rows0, acc.at[dstv.at[j0]], add=True)
            pltpu.make_async_copy(g_hbm.at[srcv.at[j0 + 1]], rows1,
                                  sem1).wait()

            @pl.when(jj + 1 < NCH // 2)
            def _():
                pltpu.async_copy(g_hbm.at[srcv.at[j0 + 2]], rows0, sem0)

            pltpu.sync_copy(rows1, acc.at[dstv.at[j0 + 1]], add=True)
            return 0
        lax.fori_loop(0, NCH // 2, chunk2, 0)
        plsc.subcore_barrier()
        pltpu.sync_copy(acc.at[pl.ds(s * RT, RT)], out_hbm.at[c, pl.ds(s * RT, RT)])

    return k(g, src_r, dst_r)


def _pairgather_sc(ta, tb, u_r, v_r):
    @functools.partial(
        pl.kernel, mesh=_mesh,
        compiler_params=pltpu.CompilerParams(use_tc_tiling_on_sc=False),
        out_type=(jax.ShapeDtypeStruct((PPAD, 16), jnp.float32),
                  jax.ShapeDtypeStruct((PPAD, 16), jnp.float32)),
        scratch_types=[
            pltpu.VMEM((PNCH, PCH), jnp.int32),
            pltpu.VMEM((PNCH, PCH), jnp.int32),
            pltpu.VMEM((PCH, 16), jnp.float32),
            pltpu.VMEM((PCH, 16), jnp.float32),
            pltpu.VMEM((PCH, 16), jnp.float32),
            pltpu.VMEM((PCH, 16), jnp.float32),
            pltpu.SemaphoreType.DMA,
            pltpu.SemaphoreType.DMA,
            pltpu.SemaphoreType.DMA,
            pltpu.SemaphoreType.DMA,
        ],
    )
    def k(ta_hbm, tb_hbm, ur_hbm, vr_hbm, outa_hbm, outb_hbm, uv, vv,
          bufa0, bufb0, bufa1, bufb1, sa0, sb0, sa1, sb1):
        c = lax.axis_index("c")
        s = lax.axis_index("s")
        wid = s * 2 + c
        pltpu.sync_copy(ur_hbm.at[wid], uv)
        pltpu.sync_copy(vr_hbm.at[wid], vv)

        pltpu.async_copy(ta_hbm.at[uv.at[0]], bufa0, sa0)
        pltpu.async_copy(tb_hbm.at[vv.at[0]], bufb0, sb0)
        pltpu.async_copy(ta_hbm.at[uv.at[1]], bufa1, sa1)
        pltpu.async_copy(tb_hbm.at[vv.at[1]], bufb1, sb1)

        def chunk2(jj, _):
            j0 = 2 * jj
            for (j, ba, bb, sba, sbb) in ((j0, bufa0, bufb0, sa0, sb0),
                                          (j0 + 1, bufa1, bufb1, sa1, sb1)):
                base = (wid * PNCH + j) * PCH
                pltpu.make_async_copy(ta_hbm.at[uv.at[j]], ba, sba).wait()
                pltpu.make_async_copy(tb_hbm.at[vv.at[j]], bb, sbb).wait()
                pltpu.sync_copy(ba, outa_hbm.at[pl.ds(base, PCH)])
                pltpu.sync_copy(bb, outb_hbm.at[pl.ds(base, PCH)])

                @pl.when(j + 2 < PNCH)
                def _():
                    pltpu.async_copy(ta_hbm.at[uv.at[j + 2]], ba, sba)
                    pltpu.async_copy(tb_hbm.at[vv.at[j + 2]], bb, sbb)
            return 0
        lax.fori_loop(0, PNCH // 2, chunk2, 0)

    return k(ta, tb, u_r, v_r)


_BLK = 1000


def _tc_first(x, w1, d0, d1):
    def body(x_ref, w_ref, d0_ref, d1_ref, ga_ref, gb_ref, dis_ref):
        deg = d0_ref[:, 0:1] + d1_ref[:, 0:1] + 1.0
        dis = lax.rsqrt(deg)
        h = jnp.dot(x_ref[...], w_ref[...], preferred_element_type=jnp.float32)
        g = dis * h
        ga_ref[...] = g[:, :64]
        gb_ref[...] = g[:, 64:]
        dis_ref[...] = jnp.broadcast_to(dis, (_BLK, 16))

    return pl.pallas_call(
        body,
        grid=(N // _BLK,),
        in_specs=[pl.BlockSpec((_BLK, 128), lambda i: (i, 0)),
                  pl.BlockSpec((128, 128), lambda i: (0, 0)),
                  pl.BlockSpec((_BLK, 16), lambda i: (i, 0)),
                  pl.BlockSpec((_BLK, 16), lambda i: (i, 0))],
        out_specs=[pl.BlockSpec((_BLK, 64), lambda i: (i, 0)),
                   pl.BlockSpec((_BLK, 64), lambda i: (i, 0)),
                   pl.BlockSpec((_BLK, 16), lambda i: (i, 0))],
        out_shape=[jax.ShapeDtypeStruct((N, 64), jnp.float32),
                   jax.ShapeDtypeStruct((N, 64), jnp.float32),
                   jax.ShapeDtypeStruct((N, 16), jnp.float32)],
    )(x, w1, d0, d1)


def _tc_mid2(p0a, p1a, p0b, p1b, ga, gb, dis16, ba_row, bb_row, wa, wb):
    def body(p0a_ref, p1a_ref, p0b_ref, p1b_ref, ga_ref, gb_ref, dis_ref,
             ba_ref, bb_ref, wa_ref, wb_ref, out_ref):
        dis = dis_ref[:, 0:1]
        t_a = jnp.maximum(
            dis * (p0a_ref[...] + p1a_ref[...] + ga_ref[...]) + ba_ref[...],
            0.0)
        t_b = jnp.maximum(
            dis * (p0b_ref[...] + p1b_ref[...] + gb_ref[...]) + bb_ref[...],
            0.0)
        out_ref[...] = dis * (
            jnp.dot(t_a, wa_ref[...], preferred_element_type=jnp.float32)
            + jnp.dot(t_b, wb_ref[...], preferred_element_type=jnp.float32))

    blk64 = pl.BlockSpec((_BLK, 64), lambda i: (i, 0))
    return pl.pallas_call(
        body,
        grid=(N // _BLK,),
        in_specs=[blk64, blk64, blk64, blk64, blk64, blk64,
                  pl.BlockSpec((_BLK, 16), lambda i: (i, 0)),
                  pl.BlockSpec((1, 64), lambda i: (0, 0)),
                  pl.BlockSpec((1, 64), lambda i: (0, 0)),
                  pl.BlockSpec((64, 64), lambda i: (0, 0)),
                  pl.BlockSpec((64, 64), lambda i: (0, 0))],
        out_specs=pl.BlockSpec((_BLK, 64), lambda i: (i, 0)),
        out_shape=jax.ShapeDtypeStruct((N, 64), jnp.float32),
    )(p0a, p1a, p0b, p1b, ga, gb, dis16, ba_row, bb_row, wa, wb)


def _tc_mid(p0, p1, g, dis16, b_row, w):
    din = g.shape[1]
    dout = w.shape[1]

    def body(p0_ref, p1_ref, g_ref, dis_ref, b_ref, w_ref, out_ref):
        dis = dis_ref[:, 0:1]
        t = dis * (p0_ref[...] + p1_ref[...] + g_ref[...]) + b_ref[...]
        t = jnp.maximum(t, 0.0)
        out_ref[...] = dis * jnp.dot(t, w_ref[...],
                                     preferred_element_type=jnp.float32)

    return pl.pallas_call(
        body,
        grid=(N // _BLK,),
        in_specs=[pl.BlockSpec((_BLK, din), lambda i: (i, 0)),
                  pl.BlockSpec((_BLK, din), lambda i: (i, 0)),
                  pl.BlockSpec((_BLK, din), lambda i: (i, 0)),
                  pl.BlockSpec((_BLK, 16), lambda i: (i, 0)),
                  pl.BlockSpec((1, din), lambda i: (0, 0)),
                  pl.BlockSpec((din, dout), lambda i: (0, 0))],
        out_specs=pl.BlockSpec((_BLK, dout), lambda i: (i, 0)),
        out_shape=jax.ShapeDtypeStruct((N, dout), jnp.float32),
    )(p0, p1, g, dis16, b_row, w)


def _tc_head(p0, p1, g3, dis16, b3_row, l1, lb1_row, l2, lb2_row, m1, mb1_row):
    def body(p0_ref, p1_ref, g_ref, dis_ref, b3_ref, l1_ref, lb1_ref,
             l2_ref, lb2_ref, m1_ref, mb1_ref, ta_ref, tb_ref):
        dis = dis_ref[:, 0:1]
        o = dis * (p0_ref[...] + p1_ref[...] + g_ref[...]) + b3_ref[...]
        o = jnp.maximum(o, 0.0)
        h4 = jnp.maximum(
            jnp.dot(o, l1_ref[...], preferred_element_type=jnp.float32)
            + lb1_ref[...], 0.0)
        emb = jnp.maximum(
            jnp.dot(h4, l2_ref[...], preferred_element_type=jnp.float32)
            + lb2_ref[...], 0.0)
        m1 = m1_ref[...]
        a = jnp.dot(emb, m1[:16, :], preferred_element_type=jnp.float32) \
            + mb1_ref[...]
        b = jnp.dot(emb, m1[16:, :], preferred_element_type=jnp.float32)
        ta_ref[...] = jnp.concatenate([a, a], axis=1)
        tb_ref[...] = jnp.concatenate([b, b], axis=1)

    return pl.pallas_call(
        body,
        grid=(N // _BLK,),
        in_specs=[pl.BlockSpec((_BLK, 32), lambda i: (i, 0)),
                  pl.BlockSpec((_BLK, 32), lambda i: (i, 0)),
                  pl.BlockSpec((_BLK, 32), lambda i: (i, 0)),
                  pl.BlockSpec((_BLK, 16), lambda i: (i, 0)),
                  pl.BlockSpec((1, 32), lambda i: (0, 0)),
                  pl.BlockSpec((32, 16), lambda i: (0, 0)),
                  pl.BlockSpec((1, 16), lambda i: (0, 0)),
                  pl.BlockSpec((16, 16), lambda i: (0, 0)),
                  pl.BlockSpec((1, 16), lambda i: (0, 0)),
                  pl.BlockSpec((32, 8), lambda i: (0, 0)),
                  pl.BlockSpec((1, 8), lambda i: (0, 0))],
        out_specs=[pl.BlockSpec((_BLK, 16), lambda i: (i, 0)),
                   pl.BlockSpec((_BLK, 16), lambda i: (i, 0))],
        out_shape=[jax.ShapeDtypeStruct((N, 16), jnp.float32),
                   jax.ShapeDtypeStruct((N, 16), jnp.float32)],
    )(p0, p1, g3, dis16, b3_row, l1, lb1_row, l2, lb2_row, m1, mb1_row)


def _tc_final(ga2, gb2, sel, mb2_s):
    rows = PPAD // 128
    blk = 200

    def body(a_ref, b_ref, sel_ref, mb2_ref, out_ref):
        t = jnp.maximum(a_ref[...] + b_ref[...], 0.0)
        sc = jnp.dot(t, sel_ref[...],
                     preferred_element_type=jnp.float32) + mb2_ref[...]
        out_ref[...] = 1.0 / (1.0 + jnp.exp(-sc))

    return pl.pallas_call(
        body,
        grid=(rows // blk,),
        in_specs=[pl.BlockSpec((blk, 2048), lambda i: (i, 0)),
                  pl.BlockSpec((blk, 2048), lambda i: (i, 0)),
                  pl.BlockSpec((2048, 128), lambda i: (0, 0)),
                  pl.BlockSpec((1, 1), lambda i: (0, 0))],
        out_specs=pl.BlockSpec((blk, 128), lambda i: (i, 0)),
        out_shape=jax.ShapeDtypeStruct((rows, 128), jnp.float32),
    )(ga2, gb2, sel, mb2_s)


def kernel(x, edge_index, pred_edges, W1, b1, W2, b2, W3, b3,
           L1, lb1, L2, lb2, M1, mb1, M2, mb2):
    ei = edge_index.astype(jnp.int32)
    src_r = ei[0].reshape(NW, NCH, CH)
    dst_r = ei[1].reshape(NW, NCH, CH)
    pe = pred_edges.astype(jnp.int32)
    u_r = jnp.pad(pe[:, 0], (0, PPAD - P)).reshape(NW, PNCH, PCH)
    v_r = jnp.pad(pe[:, 1], (0, PPAD - P)).reshape(NW, PNCH, PCH)

    degp = _deg_sc(dst_r)
    g1a, g1b, dis16 = _tc_first(x, W1, degp[0, :N], degp[1, :N])

    ppa = _scatter_sc(g1a, src_r, dst_r)
    ppb = _scatter_sc(g1b, src_r, dst_r)
    g2 = _tc_mid2(ppa[0, :N], ppa[1, :N], ppb[0, :N], ppb[1, :N],
                  g1a, g1b, dis16,
                  b1[:64].reshape(1, -1), b1[64:].reshape(1, -1),
                  W2[:64], W2[64:])

    pp = _scatter_sc(g2, src_r, dst_r)
    g3 = _tc_mid(pp[0, :N], pp[1, :N], g2, dis16, b2.reshape(1, -1), W3)

    pp = _scatter_sc(g3, src_r, dst_r)
    ta, tb = _tc_head(pp[0, :N], pp[1, :N], g3, dis16, b3.reshape(1, -1),
                      L1, lb1.reshape(1, -1), L2, lb2.reshape(1, -1),
                      M1, mb1.reshape(1, -1))

    ga, gb = _pairgather_sc(ta, tb, u_r, v_r)
    m2_pat = jnp.concatenate([M2[:, 0], jnp.zeros((8,), jnp.float32)])
    sel = jnp.kron(jnp.eye(128, dtype=jnp.float32), m2_pat.reshape(16, 1))
    y = _tc_final(ga.reshape(PPAD // 128, 2048), gb.reshape(PPAD // 128, 2048),
                  sel, mb2.reshape(1, 1))
    return y.reshape(-1)[:P]


# speedup vs baseline: 21.5556x; 1.0219x over previous
import functools

import jax
import jax.numpy as jnp
from jax import lax
from jax.experimental import pallas as pl
from jax.experimental.pallas import tpu as pltpu
from jax.experimental.pallas import tpu_sc as plsc

N = 10000
E = 320000
P = 100000
NW = 32
EW = E // NW
CH = 128
NCH = 80
EPADW = NCH * CH
EPAD = NW * EPADW
NPAD = 10240
RT = NPAD // 16
ZB = 128
PCH = 128
PNCH = 26
PPAD = NW * PNCH * PCH

_mesh = plsc.VectorSubcoreMesh(core_axis_name="c", subcore_axis_name="s")


def _deg_sc(dst_r):
    @functools.partial(
        pl.kernel, mesh=_mesh,
        compiler_params=pltpu.CompilerParams(use_tc_tiling_on_sc=False),
        out_type=jax.ShapeDtypeStruct((2, NPAD, 16), jnp.float32),
        scratch_types=[
            pltpu.VMEM((NCH, CH), jnp.int32),
            pltpu.VMEM((CH, 16), jnp.float32),
            pltpu.VMEM((ZB, 16), jnp.float32),
            pltpu.VMEM_SHARED((NPAD, 16), jnp.float32),
        ],
    )
    def k(dstr_hbm, out_hbm, dstv, ones_v, zero_v, acc):
        c = lax.axis_index("c")
        s = lax.axis_index("s")
        wid = s * 2 + c
        pltpu.sync_copy(dstr_hbm.at[wid], dstv)

        def fill(i, _):
            ones_v[i] = jnp.full((16,), 1.0, jnp.float32)
            return 0
        lax.fori_loop(0, CH, fill, 0)

        def zfill(i, _):
            zero_v[i] = jnp.zeros((16,), jnp.float32)
            return 0
        lax.fori_loop(0, ZB, zfill, 0)
        for z in range(RT // ZB):
            pltpu.sync_copy(zero_v, acc.at[pl.ds(s * RT + z * ZB, ZB)])
        plsc.subcore_barrier()

        def chunk(j, _):
            pltpu.sync_copy(ones_v, acc.at[dstv.at[j]], add=True)
            return 0
        lax.fori_loop(0, NCH, chunk, 0)
        plsc.subcore_barrier()
        pltpu.sync_copy(acc.at[pl.ds(s * RT, RT)], out_hbm.at[c, pl.ds(s * RT, RT)])

    return k(dst_r)


def _scatter_sc(g, src_r, dst_r):
    D = g.shape[1]

    @functools.partial(
        pl.kernel, mesh=_mesh,
        compiler_params=pltpu.CompilerParams(use_tc_tiling_on_sc=False),
        out_type=jax.ShapeDtypeStruct((2, NPAD, D), jnp.float32),
        scratch_types=[
            pltpu.VMEM((NCH, CH), jnp.int32),
            pltpu.VMEM((NCH, CH), jnp.int32),
            pltpu.VMEM((CH, D), jnp.float32),
            pltpu.VMEM((CH, D), jnp.float32),
            pltpu.VMEM((ZB, D), jnp.float32),
            pltpu.VMEM_SHARED((NPAD, D), jnp.float32),
            pltpu.SemaphoreType.DMA,
            pltpu.SemaphoreType.DMA,
        ],
    )
    def k(g_hbm, srcr_hbm, dstr_hbm, out_hbm, srcv, dstv, rows0, rows1,
          zero_v, acc, sem0, sem1):
        c = lax.axis_index("c")
        s = lax.axis_index("s")
        wid = s * 2 + c
        pltpu.sync_copy(srcr_hbm.at[wid], srcv)
        pltpu.sync_copy(dstr_hbm.at[wid], dstv)

        nsub = D // 16

        def zrow(t, _):
            zero_v[t // nsub, pl.ds((t % nsub) * 16, 16)] = jnp.zeros(
                (16,), jnp.float32)
            return 0
        lax.fori_loop(0, ZB * nsub, zrow, 0)
        for z in range(RT // ZB):
            pltpu.sync_copy(zero_v, acc.at[pl.ds(s * RT + z * ZB, ZB)])
        plsc.subcore_barrier()

        pltpu.async_copy(g_hbm.at[srcv.at[0]], rows0, sem0)

        def chunk2(jj, _):
            j0 = 2 * jj
            pltpu.make_async_copy(g_hbm.at[srcv.at[j0]], rows0, sem0).wait()
            pltpu.async_copy(g_hbm.at[srcv.at[j0 + 1]], rows1, sem1)
            pltpu.sync_copy(rows0, acc.at[dstv.at[j0]], add=True)
            pltpu.make_async_copy(g_hbm.at[srcv.at[j0 + 1]], rows1,
                                  sem1).wait()

            @pl.when(jj + 1 < NCH // 2)
            def _():
                pltpu.async_copy(g_hbm.at[srcv.at[j0 + 2]], rows0, sem0)

            pltpu.sync_copy(rows1, acc.at[dstv.at[j0 + 1]], add=True)
            return 0
        lax.fori_loop(0, NCH // 2, chunk2, 0)
        plsc.subcore_barrier()
        pltpu.sync_copy(acc.at[pl.ds(s * RT, RT)], out_hbm.at[c, pl.ds(s * RT, RT)])

    return k(g, src_r, dst_r)


def _pairgather_sc(ta, tb, u_r, v_r):
    @functools.partial(
        pl.kernel, mesh=_mesh,
        compiler_params=pltpu.CompilerParams(use_tc_tiling_on_sc=False),
        out_type=(jax.ShapeDtypeStruct((PPAD, 16), jnp.float32),
                  jax.ShapeDtypeStruct((PPAD, 16), jnp.float32)),
        scratch_types=[
            pltpu.VMEM((PNCH, PCH), jnp.int32),
            pltpu.VMEM((PNCH, PCH), jnp.int32),
            pltpu.VMEM((PCH, 16), jnp.float32),
            pltpu.VMEM((PCH, 16), jnp.float32),
            pltpu.VMEM((PCH, 16), jnp.float32),
            pltpu.VMEM((PCH, 16), jnp.float32),
            pltpu.SemaphoreType.DMA,
            pltpu.SemaphoreType.DMA,
            pltpu.SemaphoreType.DMA,
            pltpu.SemaphoreType.DMA,
        ],
    )
    def k(ta_hbm, tb_hbm, ur_hbm, vr_hbm, outa_hbm, outb_hbm, uv, vv,
          bufa0, bufb0, bufa1, bufb1, sa0, sb0, sa1, sb1):
        c = lax.axis_index("c")
        s = lax.axis_index("s")
        wid = s * 2 + c
        pltpu.sync_copy(ur_hbm.at[wid], uv)
        pltpu.sync_copy(vr_hbm.at[wid], vv)

        pltpu.async_copy(ta_hbm.at[uv.at[0]], bufa0, sa0)
        pltpu.async_copy(tb_hbm.at[vv.at[0]], bufb0, sb0)
        pltpu.async_copy(ta_hbm.at[uv.at[1]], bufa1, sa1)
        pltpu.async_copy(tb_hbm.at[vv.at[1]], bufb1, sb1)

        def chunk2(jj, _):
            j0 = 2 * jj
            for (j, ba, bb, sba, sbb) in ((j0, bufa0, bufb0, sa0, sb0),
                                          (j0 + 1, bufa1, bufb1, sa1, sb1)):
                base = (wid * PNCH + j) * PCH
                pltpu.make_async_copy(ta_hbm.at[uv.at[j]], ba, sba).wait()
                pltpu.make_async_copy(tb_hbm.at[vv.at[j]], bb, sbb).wait()
                pltpu.sync_copy(ba, outa_hbm.at[pl.ds(base, PCH)])
                pltpu.sync_copy(bb, outb_hbm.at[pl.ds(base, PCH)])

                @pl.when(j + 2 < PNCH)
                def _():
                    pltpu.async_copy(ta_hbm.at[uv.at[j + 2]], ba, sba)
                    pltpu.async_copy(tb_hbm.at[vv.at[j + 2]], bb, sbb)
            return 0
        lax.fori_loop(0, PNCH // 2, chunk2, 0)

    return k(ta, tb, u_r, v_r)


_BLK = 2000


def _tc_first(x, w1, d0, d1):
    def body(x_ref, w_ref, d0_ref, d1_ref, ga_ref, gb_ref, dis_ref):
        deg = d0_ref[:, 0:1] + d1_ref[:, 0:1] + 1.0
        dis = lax.rsqrt(deg)
        h = jnp.dot(x_ref[...], w_ref[...], preferred_element_type=jnp.float32)
        g = dis * h
        ga_ref[...] = g[:, :64]
        gb_ref[...] = g[:, 64:]
        dis_ref[...] = jnp.broadcast_to(dis, (_BLK, 16))

    return pl.pallas_call(
        body,
        grid=(N // _BLK,),
        in_specs=[pl.BlockSpec((_BLK, 128), lambda i: (i, 0)),
                  pl.BlockSpec((128, 128), lambda i: (0, 0)),
                  pl.BlockSpec((_BLK, 16), lambda i: (i, 0)),
                  pl.BlockSpec((_BLK, 16), lambda i: (i, 0))],
        out_specs=[pl.BlockSpec((_BLK, 64), lambda i: (i, 0)),
                   pl.BlockSpec((_BLK, 64), lambda i: (i, 0)),
                   pl.BlockSpec((_BLK, 16), lambda i: (i, 0))],
        out_shape=[jax.ShapeDtypeStruct((N, 64), jnp.float32),
                   jax.ShapeDtypeStruct((N, 64), jnp.float32),
                   jax.ShapeDtypeStruct((N, 16), jnp.float32)],
    )(x, w1, d0, d1)


def _tc_mid2(p0a, p1a, p0b, p1b, ga, gb, dis16, ba_row, bb_row, wa, wb):
    def body(p0a_ref, p1a_ref, p0b_ref, p1b_ref, ga_ref, gb_ref, dis_ref,
             ba_ref, bb_ref, wa_ref, wb_ref, out_ref):
        dis = dis_ref[:, 0:1]
        t_a = jnp.maximum(
            dis * (p0a_ref[...] + p1a_ref[...] + ga_ref[...]) + ba_ref[...],
            0.0)
        t_b = jnp.maximum(
            dis * (p0b_ref[...] + p1b_ref[...] + gb_ref[...]) + bb_ref[...],
            0.0)
        out_ref[...] = dis * (
            jnp.dot(t_a, wa_ref[...], preferred_element_type=jnp.float32)
            + jnp.dot(t_b, wb_ref[...], preferred_element_type=jnp.float32))

    blk64 = pl.BlockSpec((_BLK, 64), lambda i: (i, 0))
    return pl.pallas_call(
        body,
        grid=(N // _BLK,),
        in_specs=[blk64, blk64, blk64, blk64, blk64, blk64,
                  pl.BlockSpec((_BLK, 16), lambda i: (i, 0)),
                  pl.BlockSpec((1, 64), lambda i: (0, 0)),
                  pl.BlockSpec((1, 64), lambda i: (0, 0)),
                  pl.BlockSpec((64, 64), lambda i: (0, 0)),
                  pl.BlockSpec((64, 64), lambda i: (0, 0))],
        out_specs=pl.BlockSpec((_BLK, 64), lambda i: (i, 0)),
        out_shape=jax.ShapeDtypeStruct((N, 64), jnp.float32),
    )(p0a, p1a, p0b, p1b, ga, gb, dis16, ba_row, bb_row, wa, wb)


def _tc_mid(p0, p1, g, dis16, b_row, w):
    din = g.shape[1]
    dout = w.shape[1]

    def body(p0_ref, p1_ref, g_ref, dis_ref, b_ref, w_ref, out_ref):
        dis = dis_ref[:, 0:1]
        t = dis * (p0_ref[...] + p1_ref[...] + g_ref[...]) + b_ref[...]
        t = jnp.maximum(t, 0.0)
        out_ref[...] = dis * jnp.dot(t, w_ref[...],
                                     preferred_element_type=jnp.float32)

    return pl.pallas_call(
        body,
        grid=(N // _BLK,),
        in_specs=[pl.BlockSpec((_BLK, din), lambda i: (i, 0)),
                  pl.BlockSpec((_BLK, din), lambda i: (i, 0)),
                  pl.BlockSpec((_BLK, din), lambda i: (i, 0)),
                  pl.BlockSpec((_BLK, 16), lambda i: (i, 0)),
                  pl.BlockSpec((1, din), lambda i: (0, 0)),
                  pl.BlockSpec((din, dout), lambda i: (0, 0))],
        out_specs=pl.BlockSpec((_BLK, dout), lambda i: (i, 0)),
        out_shape=jax.ShapeDtypeStruct((N, dout), jnp.float32),
    )(p0, p1, g, dis16, b_row, w)


def _tc_head(p0, p1, g3, dis16, b3_row, l1, lb1_row, l2, lb2_row, m1, mb1_row):
    def body(p0_ref, p1_ref, g_ref, dis_ref, b3_ref, l1_ref, lb1_ref,
             l2_ref, lb2_ref, m1_ref, mb1_ref, ta_ref, tb_ref):
        dis = dis_ref[:, 0:1]
        o = dis * (p0_ref[...] + p1_ref[...] + g_ref[...]) + b3_ref[...]
        o = jnp.maximum(o, 0.0)
        h4 = jnp.maximum(
            jnp.dot(o, l1_ref[...], preferred_element_type=jnp.float32)
            + lb1_ref[...], 0.0)
        emb = jnp.maximum(
            jnp.dot(h4, l2_ref[...], preferred_element_type=jnp.float32)
            + lb2_ref[...], 0.0)
        m1 = m1_ref[...]
        a = jnp.dot(emb, m1[:16, :], preferred_element_type=jnp.float32) \
            + mb1_ref[...]
        b = jnp.dot(emb, m1[16:, :], preferred_element_type=jnp.float32)
        ta_ref[...] = jnp.concatenate([a, a], axis=1)
        tb_ref[...] = jnp.concatenate([b, b], axis=1)

    return pl.pallas_call(
        body,
        grid=(N // _BLK,),
        in_specs=[pl.BlockSpec((_BLK, 32), lambda i: (i, 0)),
                  pl.BlockSpec((_BLK, 32), lambda i: (i, 0)),
                  pl.BlockSpec((_BLK, 32), lambda i: (i, 0)),
                  pl.BlockSpec((_BLK, 16), lambda i: (i, 0)),
                  pl.BlockSpec((1, 32), lambda i: (0, 0)),
                  pl.BlockSpec((32, 16), lambda i: (0, 0)),
                  pl.BlockSpec((1, 16), lambda i: (0, 0)),
                  pl.BlockSpec((16, 16), lambda i: (0, 0)),
                  pl.BlockSpec((1, 16), lambda i: (0, 0)),
                  pl.BlockSpec((32, 8), lambda i: (0, 0)),
                  pl.BlockSpec((1, 8), lambda i: (0, 0))],
        out_specs=[pl.BlockSpec((_BLK, 16), lambda i: (i, 0)),
                   pl.BlockSpec((_BLK, 16), lambda i: (i, 0))],
        out_shape=[jax.ShapeDtypeStruct((N, 16), jnp.float32),
                   jax.ShapeDtypeStruct((N, 16), jnp.float32)],
    )(p0, p1, g3, dis16, b3_row, l1, lb1_row, l2, lb2_row, m1, mb1_row)


def _tc_final(ga2, gb2, sel, mb2_s):
    rows = PPAD // 128
    blk = 104

    def body(a_ref, b_ref, sel_ref, mb2_ref, out_ref):
        t = jnp.maximum(a_ref[...] + b_ref[...], 0.0)
        sc = jnp.dot(t, sel_ref[...],
                     preferred_element_type=jnp.float32) + mb2_ref[...]
        out_ref[...] = 1.0 / (1.0 + jnp.exp(-sc))

    return pl.pallas_call(
        body,
        grid=(rows // blk,),
        in_specs=[pl.BlockSpec((blk, 2048), lambda i: (i, 0)),
                  pl.BlockSpec((blk, 2048), lambda i: (i, 0)),
                  pl.BlockSpec((2048, 128), lambda i: (0, 0)),
                  pl.BlockSpec((1, 1), lambda i: (0, 0))],
        out_specs=pl.BlockSpec((blk, 128), lambda i: (i, 0)),
        out_shape=jax.ShapeDtypeStruct((rows, 128), jnp.float32),
    )(ga2, gb2, sel, mb2_s)


def kernel(x, edge_index, pred_edges, W1, b1, W2, b2, W3, b3,
           L1, lb1, L2, lb2, M1, mb1, M2, mb2):
    ei = edge_index.astype(jnp.int32)
    npade = EPAD - E
    pad_src = jnp.arange(npade, dtype=jnp.int32) % N
    pad_dst = N + jnp.arange(npade, dtype=jnp.int32) % (NPAD - N)
    src_r = jnp.concatenate([ei[0], pad_src]).reshape(NW, NCH, CH)
    dst_r = jnp.concatenate([ei[1], pad_dst]).reshape(NW, NCH, CH)
    pe = pred_edges.astype(jnp.int32)
    npadp = PPAD - P
    pad_p = jnp.arange(npadp, dtype=jnp.int32) % N
    u_r = jnp.concatenate([pe[:, 0], pad_p]).reshape(NW, PNCH, PCH)
    v_r = jnp.concatenate([pe[:, 1], pad_p]).reshape(NW, PNCH, PCH)

    degp = _deg_sc(dst_r)
    g1a, g1b, dis16 = _tc_first(x, W1, degp[0, :N], degp[1, :N])

    ppa = _scatter_sc(g1a, src_r, dst_r)
    ppb = _scatter_sc(g1b, src_r, dst_r)
    g2 = _tc_mid2(ppa[0, :N], ppa[1, :N], ppb[0, :N], ppb[1, :N],
                  g1a, g1b, dis16,
                  b1[:64].reshape(1, -1), b1[64:].reshape(1, -1),
                  W2[:64], W2[64:])

    pp = _scatter_sc(g2, src_r, dst_r)
    g3 = _tc_mid(pp[0, :N], pp[1, :N], g2, dis16, b2.reshape(1, -1), W3)

    pp = _scatter_sc(g3, src_r, dst_r)
    ta, tb = _tc_head(pp[0, :N], pp[1, :N], g3, dis16, b3.reshape(1, -1),
                      L1, lb1.reshape(1, -1), L2, lb2.reshape(1, -1),
                      M1, mb1.reshape(1, -1))

    ga, gb = _pairgather_sc(ta, tb, u_r, v_r)
    m2_pat = jnp.concatenate([M2[:, 0], jnp.zeros((8,), jnp.float32)])
    sel = jnp.kron(jnp.eye(128, dtype=jnp.float32), m2_pat.reshape(16, 1))
    y = _tc_final(ga.reshape(PPAD // 128, 2048), gb.reshape(PPAD // 128, 2048),
                  sel, mb2.reshape(1, 1))
    return y.reshape(-1)[:P]


# speedup vs baseline: 24.1666x; 1.1211x over previous
import functools

import jax
import jax.numpy as jnp
from jax import lax
from jax.experimental import pallas as pl
from jax.experimental.pallas import tpu as pltpu
from jax.experimental.pallas import tpu_sc as plsc

N = 10000
E = 320000
P = 100000
NW = 32
EW = E // NW
CH = 128
NCH = 80
EPADW = NCH * CH
EPAD = NW * EPADW
NPAD = 10240
RT = NPAD // 16
ZB = 128
PCH = 128
PNCH = 26
PPAD = NW * PNCH * PCH

_mesh = plsc.VectorSubcoreMesh(core_axis_name="c", subcore_axis_name="s")


def _deg_sc(dst_r):
    @functools.partial(
        pl.kernel, mesh=_mesh,
        compiler_params=pltpu.CompilerParams(use_tc_tiling_on_sc=False),
        out_type=jax.ShapeDtypeStruct((2, NPAD, 128), jnp.float32),
        scratch_types=[
            pltpu.VMEM((NCH, CH), jnp.int32),
            pltpu.VMEM((CH, 16), jnp.float32),
            pltpu.VMEM((ZB, 16), jnp.float32),
            pltpu.VMEM_SHARED((NPAD, 16), jnp.float32),
        ],
    )
    def k(dstr_hbm, out_hbm, dstv, ones_v, zero_v, acc):
        c = lax.axis_index("c")
        s = lax.axis_index("s")
        wid = s * 2 + c
        pltpu.sync_copy(dstr_hbm.at[wid], dstv)

        def fill(i, _):
            ones_v[i] = jnp.full((16,), 1.0, jnp.float32)
            return 0
        lax.fori_loop(0, CH, fill, 0)

        def zfill(i, _):
            zero_v[i] = jnp.zeros((16,), jnp.float32)
            return 0
        lax.fori_loop(0, ZB, zfill, 0)
        for z in range(RT // ZB):
            pltpu.sync_copy(zero_v, acc.at[pl.ds(s * RT + z * ZB, ZB)])
        plsc.subcore_barrier()

        def chunk(j, _):
            pltpu.sync_copy(ones_v, acc.at[dstv.at[j]], add=True)
            return 0
        lax.fori_loop(0, NCH, chunk, 0)
        plsc.subcore_barrier()
        pltpu.sync_copy(acc.at[pl.ds(s * RT, RT)],
                        out_hbm.at[c, pl.ds(s * RT, RT), pl.ds(0, 16)])

    return k(dst_r)


def _scatter_sc(g, src_r, dst_r):
    D = g.shape[1]

    @functools.partial(
        pl.kernel, mesh=_mesh,
        compiler_params=pltpu.CompilerParams(use_tc_tiling_on_sc=False),
        out_type=jax.ShapeDtypeStruct((2, NPAD, 128), jnp.float32),
        scratch_types=[
            pltpu.VMEM((NCH, CH), jnp.int32),
            pltpu.VMEM((NCH, CH), jnp.int32),
            pltpu.VMEM((CH, D), jnp.float32),
            pltpu.VMEM((CH, D), jnp.float32),
            pltpu.VMEM((ZB, D), jnp.float32),
            pltpu.VMEM_SHARED((NPAD, D), jnp.float32),
            pltpu.SemaphoreType.DMA,
            pltpu.SemaphoreType.DMA,
        ],
    )
    def k(g_hbm, srcr_hbm, dstr_hbm, out_hbm, srcv, dstv, rows0, rows1,
          zero_v, acc, sem0, sem1):
        c = lax.axis_index("c")
        s = lax.axis_index("s")
        wid = s * 2 + c
        pltpu.sync_copy(srcr_hbm.at[wid], srcv)
        pltpu.sync_copy(dstr_hbm.at[wid], dstv)

        nsub = D // 16

        def zrow(t, _):
            zero_v[t // nsub, pl.ds((t % nsub) * 16, 16)] = jnp.zeros(
                (16,), jnp.float32)
            return 0
        lax.fori_loop(0, ZB * nsub, zrow, 0)
        for z in range(RT // ZB):
            pltpu.sync_copy(zero_v, acc.at[pl.ds(s * RT + z * ZB, ZB)])
        plsc.subcore_barrier()

        pltpu.async_copy(g_hbm.at[srcv.at[0]], rows0, sem0)

        def chunk2(jj, _):
            j0 = 2 * jj
            pltpu.make_async_copy(g_hbm.at[srcv.at[j0]], rows0, sem0).wait()
            pltpu.async_copy(g_hbm.at[srcv.at[j0 + 1]], rows1, sem1)
            pltpu.sync_copy(rows0, acc.at[dstv.at[j0]], add=True)
            pltpu.make_async_copy(g_hbm.at[srcv.at[j0 + 1]], rows1,
                                  sem1).wait()

            @pl.when(jj + 1 < NCH // 2)
            def _():
                pltpu.async_copy(g_hbm.at[srcv.at[j0 + 2]], rows0, sem0)

            pltpu.sync_copy(rows1, acc.at[dstv.at[j0 + 1]], add=True)
            return 0
        lax.fori_loop(0, NCH // 2, chunk2, 0)
        plsc.subcore_barrier()
        pltpu.sync_copy(acc.at[pl.ds(s * RT, RT)],
                        out_hbm.at[c, pl.ds(s * RT, RT), pl.ds(0, D)])

    return k(g, src_r, dst_r)


def _pairgather_sc(ta, tb, u_r, v_r):
    @functools.partial(
        pl.kernel, mesh=_mesh,
        compiler_params=pltpu.CompilerParams(use_tc_tiling_on_sc=False),
        out_type=(jax.ShapeDtypeStruct((PPAD, 16), jnp.float32),
                  jax.ShapeDtypeStruct((PPAD, 16), jnp.float32)),
        scratch_types=[
            pltpu.VMEM((PNCH, PCH), jnp.int32),
            pltpu.VMEM((PNCH, PCH), jnp.int32),
            pltpu.VMEM((PCH, 16), jnp.float32),
            pltpu.VMEM((PCH, 16), jnp.float32),
            pltpu.VMEM((PCH, 16), jnp.float32),
            pltpu.VMEM((PCH, 16), jnp.float32),
            pltpu.SemaphoreType.DMA,
            pltpu.SemaphoreType.DMA,
            pltpu.SemaphoreType.DMA,
            pltpu.SemaphoreType.DMA,
        ],
    )
    def k(ta_hbm, tb_hbm, ur_hbm, vr_hbm, outa_hbm, outb_hbm, uv, vv,
          bufa0, bufb0, bufa1, bufb1, sa0, sb0, sa1, sb1):
        c = lax.axis_index("c")
        s = lax.axis_index("s")
        wid = s * 2 + c
        pltpu.sync_copy(ur_hbm.at[wid], uv)
        pltpu.sync_copy(vr_hbm.at[wid], vv)

        pltpu.async_copy(ta_hbm.at[uv.at[0]], bufa0, sa0)
        pltpu.async_copy(tb_hbm.at[vv.at[0]], bufb0, sb0)
        pltpu.async_copy(ta_hbm.at[uv.at[1]], bufa1, sa1)
        pltpu.async_copy(tb_hbm.at[vv.at[1]], bufb1, sb1)

        def chunk2(jj, _):
            j0 = 2 * jj
            for (j, ba, bb, sba, sbb) in ((j0, bufa0, bufb0, sa0, sb0),
                                          (j0 + 1, bufa1, bufb1, sa1, sb1)):
                base = (wid * PNCH + j) * PCH
                pltpu.make_async_copy(ta_hbm.at[uv.at[j]], ba, sba).wait()
                pltpu.make_async_copy(tb_hbm.at[vv.at[j]], bb, sbb).wait()
                pltpu.sync_copy(ba, outa_hbm.at[pl.ds(base, PCH)])
                pltpu.sync_copy(bb, outb_hbm.at[pl.ds(base, PCH)])

                @pl.when(j + 2 < PNCH)
                def _():
                    pltpu.async_copy(ta_hbm.at[uv.at[j + 2]], ba, sba)
                    pltpu.async_copy(tb_hbm.at[vv.at[j + 2]], bb, sbb)
            return 0
        lax.fori_loop(0, PNCH // 2, chunk2, 0)

    return k(ta, tb, u_r, v_r)


_BLK = 2000


def _tc_first(x, w1, degp):
    def body(x_ref, w_ref, d0_ref, d1_ref, ga_ref, gb_ref, dis_ref):
        deg = d0_ref[0, :, 0:1] + d1_ref[0, :, 0:1] + 1.0
        dis = lax.rsqrt(deg)
        h = jnp.dot(x_ref[...], w_ref[...], preferred_element_type=jnp.float32)
        g = dis * h
        ga_ref[...] = g[:, :64]
        gb_ref[...] = g[:, 64:]
        dis_ref[...] = jnp.broadcast_to(dis, (_BLK, 16))

    return pl.pallas_call(
        body,
        grid=(N // _BLK,),
        in_specs=[pl.BlockSpec((_BLK, 128), lambda i: (i, 0)),
                  pl.BlockSpec((128, 128), lambda i: (0, 0)),
                  pl.BlockSpec((1, _BLK, 128), lambda i: (0, i, 0)),
                  pl.BlockSpec((1, _BLK, 128), lambda i: (1, i, 0))],
        out_specs=[pl.BlockSpec((_BLK, 64), lambda i: (i, 0)),
                   pl.BlockSpec((_BLK, 64), lambda i: (i, 0)),
                   pl.BlockSpec((_BLK, 16), lambda i: (i, 0))],
        out_shape=[jax.ShapeDtypeStruct((N, 64), jnp.float32),
                   jax.ShapeDtypeStruct((N, 64), jnp.float32),
                   jax.ShapeDtypeStruct((N, 16), jnp.float32)],
    )(x, w1, degp, degp)


def _tc_mid2(ppa, ppb, ga, gb, dis16, ba_row, bb_row, wa, wb):
    def body(p0a_ref, p1a_ref, p0b_ref, p1b_ref, ga_ref, gb_ref, dis_ref,
             ba_ref, bb_ref, wa_ref, wb_ref, out_ref):
        dis = dis_ref[:, 0:1]
        t_a = jnp.maximum(
            dis * (p0a_ref[0, :, :64] + p1a_ref[0, :, :64] + ga_ref[...])
            + ba_ref[...], 0.0)
        t_b = jnp.maximum(
            dis * (p0b_ref[0, :, :64] + p1b_ref[0, :, :64] + gb_ref[...])
            + bb_ref[...], 0.0)
        out_ref[...] = dis * (
            jnp.dot(t_a, wa_ref[...], preferred_element_type=jnp.float32)
            + jnp.dot(t_b, wb_ref[...], preferred_element_type=jnp.float32))

    blk64 = pl.BlockSpec((_BLK, 64), lambda i: (i, 0))
    pblk0 = pl.BlockSpec((1, _BLK, 128), lambda i: (0, i, 0))
    pblk1 = pl.BlockSpec((1, _BLK, 128), lambda i: (1, i, 0))
    return pl.pallas_call(
        body,
        grid=(N // _BLK,),
        in_specs=[pblk0, pblk1, pblk0, pblk1, blk64, blk64,
                  pl.BlockSpec((_BLK, 16), lambda i: (i, 0)),
                  pl.BlockSpec((1, 64), lambda i: (0, 0)),
                  pl.BlockSpec((1, 64), lambda i: (0, 0)),
                  pl.BlockSpec((64, 64), lambda i: (0, 0)),
                  pl.BlockSpec((64, 64), lambda i: (0, 0))],
        out_specs=pl.BlockSpec((_BLK, 64), lambda i: (i, 0)),
        out_shape=jax.ShapeDtypeStruct((N, 64), jnp.float32),
    )(ppa, ppa, ppb, ppb, ga, gb, dis16, ba_row, bb_row, wa, wb)


def _tc_mid(pp, g, dis16, b_row, w):
    din = g.shape[1]
    dout = w.shape[1]

    def body(p0_ref, p1_ref, g_ref, dis_ref, b_ref, w_ref, out_ref):
        dis = dis_ref[:, 0:1]
        p0 = p0_ref[0, :, :din]
        p1 = p1_ref[0, :, :din]
        t = dis * (p0 + p1 + g_ref[...]) + b_ref[...]
        t = jnp.maximum(t, 0.0)
        out_ref[...] = dis * jnp.dot(t, w_ref[...],
                                     preferred_element_type=jnp.float32)

    return pl.pallas_call(
        body,
        grid=(N // _BLK,),
        in_specs=[pl.BlockSpec((1, _BLK, 128), lambda i: (0, i, 0)),
                  pl.BlockSpec((1, _BLK, 128), lambda i: (1, i, 0)),
                  pl.BlockSpec((_BLK, din), lambda i: (i, 0)),
                  pl.BlockSpec((_BLK, 16), lambda i: (i, 0)),
                  pl.BlockSpec((1, din), lambda i: (0, 0)),
                  pl.BlockSpec((din, dout), lambda i: (0, 0))],
        out_specs=pl.BlockSpec((_BLK, dout), lambda i: (i, 0)),
        out_shape=jax.ShapeDtypeStruct((N, dout), jnp.float32),
    )(pp, pp, g, dis16, b_row, w)


def _tc_head(pp, g3, dis16, b3_row, l1, lb1_row, l2, lb2_row, m1, mb1_row):
    def body(p0_ref, p1_ref, g_ref, dis_ref, b3_ref, l1_ref, lb1_ref,
             l2_ref, lb2_ref, m1_ref, mb1_ref, ta_ref, tb_ref):
        dis = dis_ref[:, 0:1]
        o = dis * (p0_ref[0, :, :32] + p1_ref[0, :, :32] + g_ref[...]) \
            + b3_ref[...]
        o = jnp.maximum(o, 0.0)
        h4 = jnp.maximum(
            jnp.dot(o, l1_ref[...], preferred_element_type=jnp.float32)
            + lb1_ref[...], 0.0)
        emb = jnp.maximum(
            jnp.dot(h4, l2_ref[...], preferred_element_type=jnp.float32)
            + lb2_ref[...], 0.0)
        m1 = m1_ref[...]
        a = jnp.dot(emb, m1[:16, :], preferred_element_type=jnp.float32) \
            + mb1_ref[...]
        b = jnp.dot(emb, m1[16:, :], preferred_element_type=jnp.float32)
        ta_ref[...] = jnp.concatenate([a, a], axis=1)
        tb_ref[...] = jnp.concatenate([b, b], axis=1)

    return pl.pallas_call(
        body,
        grid=(N // _BLK,),
        in_specs=[pl.BlockSpec((1, _BLK, 128), lambda i: (0, i, 0)),
                  pl.BlockSpec((1, _BLK, 128), lambda i: (1, i, 0)),
                  pl.BlockSpec((_BLK, 32), lambda i: (i, 0)),
                  pl.BlockSpec((_BLK, 16), lambda i: (i, 0)),
                  pl.BlockSpec((1, 32), lambda i: (0, 0)),
                  pl.BlockSpec((32, 16), lambda i: (0, 0)),
                  pl.BlockSpec((1, 16), lambda i: (0, 0)),
                  pl.BlockSpec((16, 16), lambda i: (0, 0)),
                  pl.BlockSpec((1, 16), lambda i: (0, 0)),
                  pl.BlockSpec((32, 8), lambda i: (0, 0)),
                  pl.BlockSpec((1, 8), lambda i: (0, 0))],
        out_specs=[pl.BlockSpec((_BLK, 16), lambda i: (i, 0)),
                   pl.BlockSpec((_BLK, 16), lambda i: (i, 0))],
        out_shape=[jax.ShapeDtypeStruct((N, 16), jnp.float32),
                   jax.ShapeDtypeStruct((N, 16), jnp.float32)],
    )(pp, pp, g3, dis16, b3_row, l1, lb1_row, l2, lb2_row, m1, mb1_row)


def _tc_final(ga2, gb2, sel, mb2_s):
    rows = PPAD // 128
    blk = 104

    def body(a_ref, b_ref, sel_ref, mb2_ref, out_ref):
        t = jnp.maximum(a_ref[...] + b_ref[...], 0.0)
        sc = jnp.dot(t, sel_ref[...],
                     preferred_element_type=jnp.float32) + mb2_ref[...]
        out_ref[...] = 1.0 / (1.0 + jnp.exp(-sc))

    return pl.pallas_call(
        body,
        grid=(rows // blk,),
        in_specs=[pl.BlockSpec((blk, 2048), lambda i: (i, 0)),
                  pl.BlockSpec((blk, 2048), lambda i: (i, 0)),
                  pl.BlockSpec((2048, 128), lambda i: (0, 0)),
                  pl.BlockSpec((1, 1), lambda i: (0, 0))],
        out_specs=pl.BlockSpec((blk, 128), lambda i: (i, 0)),
        out_shape=jax.ShapeDtypeStruct((rows, 128), jnp.float32),
    )(ga2, gb2, sel, mb2_s)


def kernel(x, edge_index, pred_edges, W1, b1, W2, b2, W3, b3,
           L1, lb1, L2, lb2, M1, mb1, M2, mb2):
    ei = edge_index.astype(jnp.int32)
    npade = EPAD - E
    pad_src = jnp.arange(npade, dtype=jnp.int32) % N
    pad_dst = N + jnp.arange(npade, dtype=jnp.int32) % (NPAD - N)
    src_r = jnp.concatenate([ei[0], pad_src]).reshape(NW, NCH, CH)
    dst_r = jnp.concatenate([ei[1], pad_dst]).reshape(NW, NCH, CH)
    pe = pred_edges.astype(jnp.int32)
    npadp = PPAD - P
    pad_p = jnp.arange(npadp, dtype=jnp.int32) % N
    u_r = jnp.concatenate([pe[:, 0], pad_p]).reshape(NW, PNCH, PCH)
    v_r = jnp.concatenate([pe[:, 1], pad_p]).reshape(NW, PNCH, PCH)

    degp = _deg_sc(dst_r)
    g1a, g1b, dis16 = _tc_first(x, W1, degp)

    ppa = _scatter_sc(g1a, src_r, dst_r)
    ppb = _scatter_sc(g1b, src_r, dst_r)
    g2 = _tc_mid2(ppa, ppb, g1a, g1b, dis16,
                  b1[:64].reshape(1, -1), b1[64:].reshape(1, -1),
                  W2[:64], W2[64:])

    pp = _scatter_sc(g2, src_r, dst_r)
    g3 = _tc_mid(pp, g2, dis16, b2.reshape(1, -1), W3)

    pp = _scatter_sc(g3, src_r, dst_r)
    ta, tb = _tc_head(pp, g3, dis16, b3.reshape(1, -1),
                      L1, lb1.reshape(1, -1), L2, lb2.reshape(1, -1),
                      M1, mb1.reshape(1, -1))

    ga, gb = _pairgather_sc(ta, tb, u_r, v_r)
    m2_pat = jnp.concatenate([M2[:, 0], jnp.zeros((8,), jnp.float32)])
    sel = jnp.kron(jnp.eye(128, dtype=jnp.float32), m2_pat.reshape(16, 1))
    y = _tc_final(ga.reshape(PPAD // 128, 2048), gb.reshape(PPAD // 128, 2048),
                  sel, mb2.reshape(1, 1))
    return y.reshape(-1)[:P]
